# baseline (speedup 1.0000x reference)
import functools

import jax
import jax.numpy as jnp
from jax import lax
from jax.experimental import pallas as pl
from jax.experimental.pallas import tpu as pltpu
from jax.experimental.pallas import tpu_sc as plsc

_NC = 2
_NS = 16
_CHUNK = 128
_GCH = 512
_HW = 16


def _sc_mesh():
    return plsc.VectorSubcoreMesh(core_axis_name="c", subcore_axis_name="s")


def _hist_sc(d3, n_pad, ept):
    nch = ept // _CHUNK
    nrt = n_pad // _NS

    @functools.partial(
        pl.kernel,
        out_type=jax.ShapeDtypeStruct((_NC, n_pad, _HW), jnp.float32),
        mesh=_sc_mesh(),
        compiler_params=pltpu.CompilerParams(use_tc_tiling_on_sc=False),
        scratch_types=[
            pltpu.VMEM((nch, _CHUNK), jnp.int32),
            pltpu.VMEM((_CHUNK, _HW), jnp.float32),
            pltpu.VMEM((nrt, _HW), jnp.float32),
            pltpu.VMEM_SHARED((n_pad, _HW), jnp.float32),
        ],
    )
    def k(d_hbm, out_hbm, idx_v, ones_v, buf_v, deg_sh):
        cid = lax.axis_index("c")
        sid = lax.axis_index("s")
        wid = cid * _NS + sid

        pltpu.sync_copy(d_hbm.at[wid], idx_v)

        @pl.loop(0, _CHUNK)
        def _(i):
            ones_v[pl.ds(i, 1), pl.ds(0, _HW)] = jnp.ones((1, _HW), jnp.float32)

        @pl.loop(0, nrt)
        def _(i):
            buf_v[pl.ds(i, 1), pl.ds(0, _HW)] = jnp.zeros((1, _HW), jnp.float32)

        pltpu.sync_copy(buf_v, deg_sh.at[pl.ds(sid * nrt, nrt)])
        plsc.subcore_barrier()

        @pl.loop(0, nch)
        def _(j):
            pltpu.sync_copy(ones_v, deg_sh.at[idx_v.at[j]], add=True)

        plsc.subcore_barrier()
        pltpu.sync_copy(deg_sh.at[pl.ds(sid * nrt, nrt)], buf_v)
        pltpu.sync_copy(buf_v, out_hbm.at[cid].at[pl.ds(sid * nrt, nrt)])

    return k(d3)


def _scatter_sc(y2, sd4, n_pad, ept):
    dh = y2.shape[2]
    nch = ept // _CHUNK
    nrt = n_pad // _NS

    @functools.partial(
        pl.kernel,
        out_type=jax.ShapeDtypeStruct((_NC, n_pad, dh), jnp.float32),
        mesh=_sc_mesh(),
        compiler_params=pltpu.CompilerParams(use_tc_tiling_on_sc=False),
        scratch_types=[
            pltpu.VMEM((4, 2, _CHUNK), jnp.int32),
            pltpu.VMEM((4 * _CHUNK, dh), jnp.float32),
            pltpu.VMEM_SHARED((n_pad, dh), jnp.float32),
            pltpu.VMEM_SHARED((n_pad, dh), jnp.float32),
        ] + [pltpu.SemaphoreType.DMA] * 8,
    )
    def k(y_hbm, e_hbm, out_hbm, idxr, rows, z_sh, y_sp, *sems):
        sems_i = sems[:4]
        sems_g = sems[4:]
        cid = lax.axis_index("c")
        sid = lax.axis_index("s")

        def idx_cp(c, slot):
            return pltpu.make_async_copy(e_hbm.at[sid].at[c], idxr.at[slot],
                                         sems_i[slot])

        def g_cp(g, slot):
            return pltpu.make_async_copy(
                y_sp.at[idxr.at[slot, 0]],
                rows.at[pl.ds(slot * _CHUNK, _CHUNK)], sems_g[slot])

        @pl.loop(0, _CHUNK)
        def _(i):
            @pl.loop(0, dh, step=16)
            def _(c):
                rows[pl.ds(i, 1), pl.ds(c, 16)] = jnp.zeros((1, 16),
                                                            jnp.float32)

        @pl.loop(0, nrt, step=_CHUNK)
        def _(r):
            pltpu.sync_copy(rows.at[pl.ds(0, _CHUNK)],
                            z_sh.at[pl.ds(sid * nrt + r, _CHUNK)])
            pltpu.sync_copy(y_hbm.at[cid].at[pl.ds(sid * nrt + r, _CHUNK)],
                            rows.at[pl.ds(_CHUNK, _CHUNK)])
            pltpu.sync_copy(rows.at[pl.ds(_CHUNK, _CHUNK)],
                            y_sp.at[pl.ds(sid * nrt + r, _CHUNK)])

        plsc.subcore_barrier()

        for j in range(4):
            idx_cp(j, j).start()
        idx_cp(0, 0).wait()
        g_cp(0, 0).start()

        @pl.loop(0, nch, step=4)
        def _(c0):
            for j in range(4):
                c = c0 + j
                jn = (j + 1) % 4
                g_cp(c, j).wait()
                cn = c + 1

                @pl.when(cn < nch)
                def _():
                    idx_cp(cn, jn).wait()
                    g_cp(cn, jn).start()

                pltpu.sync_copy(rows.at[pl.ds(j * _CHUNK, _CHUNK)],
                                z_sh.at[idxr.at[j, 1]], add=True)

                @pl.when(c + 4 < nch)
                def _():
                    idx_cp(c + 4, j).start()

        plsc.subcore_barrier()
        for kk in range(nrt // _CHUNK):
            off = sid * nrt + kk * _CHUNK
            pltpu.sync_copy(z_sh.at[pl.ds(off, _CHUNK)], rows.at[pl.ds(0, _CHUNK)])
            pltpu.sync_copy(rows.at[pl.ds(0, _CHUNK)],
                            out_hbm.at[cid].at[pl.ds(off, _CHUNK)])

    return k(y2, sd4)


def _dense_in_tc(x, W_in, b_in, ln1_w, ln1_b, conv_W, n, n_pad):
    d = x.shape[1]
    dh = W_in.shape[1]
    blk = 1024
    grid = n_pad // blk

    def body(x_ref, w1_ref, b1_ref, lw_ref, lb_ref, w2_ref, h_ref, xw_ref):
        i = pl.program_id(0)
        t = jnp.dot(x_ref[...], w1_ref[...], preferred_element_type=jnp.float32)
        t = t + b1_ref[...]
        mu = jnp.mean(t, axis=1, keepdims=True)
        var = jnp.mean((t - mu) ** 2, axis=1, keepdims=True)
        t = (t - mu) * lax.rsqrt(var + 1e-5) * lw_ref[...] + lb_ref[...]
        h = jnp.maximum(t, 0.0)
        rowid = i * blk + lax.broadcasted_iota(jnp.int32, (blk, 1), 0)
        h = jnp.where(rowid < n, h, 0.0)
        h_ref[...] = h
        xw_ref[...] = jnp.dot(h, w2_ref[...], preferred_element_type=jnp.float32)

    return pl.pallas_call(
        body,
        grid=(grid,),
        in_specs=[
            pl.BlockSpec((blk, d), lambda i: (i, 0)),
            pl.BlockSpec((d, dh), lambda i: (0, 0)),
            pl.BlockSpec((1, dh), lambda i: (0, 0)),
            pl.BlockSpec((1, dh), lambda i: (0, 0)),
            pl.BlockSpec((1, dh), lambda i: (0, 0)),
            pl.BlockSpec((dh, dh), lambda i: (0, 0)),
        ],
        out_specs=[pl.BlockSpec((blk, dh), lambda i: (i, 0))] * 2,
        out_shape=[jax.ShapeDtypeStruct((n_pad, dh), jnp.float32)] * 2,
    )(x, W_in.astype(jnp.float32), b_in.reshape(1, dh), ln1_w.reshape(1, dh),
      ln1_b.reshape(1, dh), conv_W.astype(jnp.float32))


def _y_tc(cnt, xw_pad, n_pad):
    d = xw_pad.shape[1]
    dh = d // 2
    blk = 1024
    grid = n_pad // blk

    def body(cnt_ref, xw_ref, y_ref, dv_ref):
        c = cnt_ref[...]
        deg = c[0, :, :1] + c[1, :, :1] + 1.0
        dinv = lax.rsqrt(deg)
        dv_ref[...] = dinv
        y = xw_ref[...] * dinv
        y_ref[...] = jnp.stack([y[:, :dh], y[:, dh:]], axis=0)

    return pl.pallas_call(
        body,
        grid=(grid,),
        in_specs=[
            pl.BlockSpec((2, blk, _HW), lambda i: (0, i, 0)),
            pl.BlockSpec((blk, d), lambda i: (i, 0)),
        ],
        out_specs=[
            pl.BlockSpec((2, blk, dh), lambda i: (0, i, 0)),
            pl.BlockSpec((blk, 1), lambda i: (i, 0)),
        ],
        out_shape=[
            jax.ShapeDtypeStruct((2, n_pad, dh), jnp.float32),
            jax.ShapeDtypeStruct((n_pad, 1), jnp.float32),
        ],
    )(cnt, xw_pad)


def _b_tc(zp, xw_pad, dinv, conv_b, h_pad, gn_w, gn_b, gn_ms, proj_W,
          proj_b, ln2_w, ln2_b, n):
    d = xw_pad.shape[1]
    dh = d // 2
    do = proj_W.shape[1]
    blk = 1000
    grid = n // blk

    def body(z_ref, xw_ref, dv_ref, cb_ref, h_ref, gw_ref, gb_ref, gms_ref,
             pw_ref, pb_ref, l2w_ref, l2b_ref, out_ref, s_acc):
        p = pl.program_id(0)
        i = pl.program_id(1)
        zs = z_ref[...]
        z = jnp.concatenate([zs[0], zs[1]], axis=1)
        dv = dv_ref[...]
        agg = dv * z + dv * dv * xw_ref[...] + cb_ref[...]

        @pl.when(p == 0)
        def _():
            s1 = jnp.sum(agg, axis=0, keepdims=True)
            s2 = jnp.sum(agg * agg, axis=0, keepdims=True)
            s = jnp.concatenate([s1, s2], axis=0)

            @pl.when(i == 0)
            def _():
                s_acc[...] = s

            @pl.when(i > 0)
            def _():
                s_acc[...] += s

        @pl.when(p == 1)
        def _():
            s_val = s_acc[...]
            mean = s_val[0:1] / n
            m = gms_ref[...] * mean
            var = s_val[1:2] / n - 2.0 * m * mean + m * m
            c = (agg - m) * lax.rsqrt(var + 1e-5) * gw_ref[...] + gb_ref[...]
            c = jnp.maximum(c, 0.0)
            hr = c + h_ref[...]
            pp = jnp.dot(hr, pw_ref[...], preferred_element_type=jnp.float32)
            pp = pp + pb_ref[...]
            mu = jnp.mean(pp, axis=1, keepdims=True)
            v2 = jnp.mean((pp - mu) ** 2, axis=1, keepdims=True)
            pp = (pp - mu) * lax.rsqrt(v2 + 1e-5) * l2w_ref[...] + l2b_ref[...]
            nrm = jnp.sqrt(jnp.sum(pp * pp, axis=1, keepdims=True))
            out_ref[...] = pp / jnp.maximum(nrm, 1e-12)

    return pl.pallas_call(
        body,
        grid=(2, grid),
        in_specs=[
            pl.BlockSpec((2, blk, dh), lambda p, i: (0, i, 0)),
            pl.BlockSpec((blk, d), lambda p, i: (i, 0)),
            pl.BlockSpec((blk, 1), lambda p, i: (i, 0)),
            pl.BlockSpec((1, d), lambda p, i: (0, 0)),
            pl.BlockSpec((blk, d), lambda p, i: (i, 0)),
            pl.BlockSpec((1, d), lambda p, i: (0, 0)),
            pl.BlockSpec((1, d), lambda p, i: (0, 0)),
            pl.BlockSpec((1, d), lambda p, i: (0, 0)),
            pl.BlockSpec((d, do), lambda p, i: (0, 0)),
            pl.BlockSpec((1, do), lambda p, i: (0, 0)),
            pl.BlockSpec((1, do), lambda p, i: (0, 0)),
            pl.BlockSpec((1, do), lambda p, i: (0, 0)),
        ],
        out_specs=pl.BlockSpec((blk, do), lambda p, i: (i, 0)),
        out_shape=jax.ShapeDtypeStruct((n, do), jnp.float32),
        scratch_shapes=[pltpu.VMEM((2, d), jnp.float32)],
    )(zp, xw_pad, dinv, conv_b.reshape(1, d), h_pad[:n], gn_w.reshape(1, d),
      gn_b.reshape(1, d), gn_ms.reshape(1, d), proj_W, proj_b.reshape(1, do),
      ln2_w.reshape(1, do), ln2_b.reshape(1, do))


def kernel(x, edge_index, W_in, b_in, ln1_w, ln1_b, conv_W, conv_b,
           gn_w, gn_b, gn_ms, proj_W, proj_b, ln2_w, ln2_b):
    n = x.shape[0]
    e = edge_index.shape[1]
    n_pad = ((n + 1023) // 1024) * 1024
    ept = ((e + _NS * _GCH - 1) // (_NS * _GCH)) * _GCH
    e_pad = ept * _NS
    nch = ept // _CHUNK

    s = edge_index[0]
    dst = edge_index[1]
    pad = e_pad - e
    s_pad = jnp.concatenate([s, jnp.full((pad,), n, jnp.int32)])
    d_pad = jnp.concatenate([dst, jnp.full((pad,), n, jnp.int32)])
    sd4 = jnp.stack([s_pad.reshape(_NS, nch, _CHUNK),
                     d_pad.reshape(_NS, nch, _CHUNK)], axis=2)

    epth = e_pad // (_NC * _NS)
    d3h = d_pad.reshape(_NC * _NS, epth // _CHUNK, _CHUNK)

    cnt_w = _hist_sc(d3h, n_pad, epth)

    h_pad, xw_pad = _dense_in_tc(x, W_in, b_in, ln1_w, ln1_b, conv_W, n, n_pad)
    y2, dinv = _y_tc(cnt_w, xw_pad, n_pad)
    zp = _scatter_sc(y2, sd4, n_pad, ept)
    return _b_tc(zp, xw_pad, dinv, conv_b, h_pad, gn_w, gn_b, gn_ms,
                 proj_W, proj_b, ln2_w, ln2_b, n)


# speedup vs baseline: 1.0233x; 1.0233x over previous
import functools

import jax
import jax.numpy as jnp
from jax import lax
from jax.experimental import pallas as pl
from jax.experimental.pallas import tpu as pltpu
from jax.experimental.pallas import tpu_sc as plsc

_NC = 2
_NS = 16
_CHUNK = 128
_GCH = 512
_HW = 16


def _sc_mesh():
    return plsc.VectorSubcoreMesh(core_axis_name="c", subcore_axis_name="s")


def _hist_sc(d3, n_pad, ept):
    nch = ept // _CHUNK
    nrt = n_pad // _NS

    @functools.partial(
        pl.kernel,
        out_type=jax.ShapeDtypeStruct((_NC, n_pad, _HW), jnp.float32),
        mesh=_sc_mesh(),
        compiler_params=pltpu.CompilerParams(use_tc_tiling_on_sc=False),
        scratch_types=[
            pltpu.VMEM((nch, _CHUNK), jnp.int32),
            pltpu.VMEM((_CHUNK, _HW), jnp.float32),
            pltpu.VMEM((nrt, _HW), jnp.float32),
            pltpu.VMEM_SHARED((n_pad, _HW), jnp.float32),
        ],
    )
    def k(d_hbm, out_hbm, idx_v, ones_v, buf_v, deg_sh):
        cid = lax.axis_index("c")
        sid = lax.axis_index("s")
        wid = cid * _NS + sid

        pltpu.sync_copy(d_hbm.at[wid], idx_v)

        @pl.loop(0, _CHUNK)
        def _(i):
            ones_v[pl.ds(i, 1), pl.ds(0, _HW)] = jnp.ones((1, _HW), jnp.float32)

        @pl.loop(0, nrt)
        def _(i):
            buf_v[pl.ds(i, 1), pl.ds(0, _HW)] = jnp.zeros((1, _HW), jnp.float32)

        pltpu.sync_copy(buf_v, deg_sh.at[pl.ds(sid * nrt, nrt)])
        plsc.subcore_barrier()

        @pl.loop(0, nch)
        def _(j):
            pltpu.sync_copy(ones_v, deg_sh.at[idx_v.at[j]], add=True)

        plsc.subcore_barrier()
        pltpu.sync_copy(deg_sh.at[pl.ds(sid * nrt, nrt)], buf_v)
        pltpu.sync_copy(buf_v, out_hbm.at[cid].at[pl.ds(sid * nrt, nrt)])

    return k(d3)


def _scatter_sc(y2, sd4, n_pad, ept):
    dh = y2.shape[2]
    nch = ept // _CHUNK
    nrt = n_pad // _NS

    @functools.partial(
        pl.kernel,
        out_type=jax.ShapeDtypeStruct((_NC, n_pad, dh), jnp.float32),
        mesh=_sc_mesh(),
        compiler_params=pltpu.CompilerParams(use_tc_tiling_on_sc=False),
        scratch_types=[
            pltpu.VMEM((4, 2, _CHUNK), jnp.int32),
            pltpu.VMEM((4 * _CHUNK, dh), jnp.float32),
            pltpu.VMEM_SHARED((n_pad, dh), jnp.float32),
            pltpu.VMEM_SHARED((n_pad, dh), jnp.float32),
        ] + [pltpu.SemaphoreType.DMA] * 8,
    )
    def k(y_hbm, e_hbm, out_hbm, idxr, rows, z_sh, y_sp, *sems):
        sems_i = sems[:4]
        sems_g = sems[4:]
        cid = lax.axis_index("c")
        sid = lax.axis_index("s")

        def idx_cp(c, slot):
            return pltpu.make_async_copy(e_hbm.at[sid].at[c], idxr.at[slot],
                                         sems_i[slot])

        def g_cp(g, slot):
            return pltpu.make_async_copy(
                y_sp.at[idxr.at[slot, 0]],
                rows.at[pl.ds(slot * _CHUNK, _CHUNK)], sems_g[slot])

        @pl.loop(0, _CHUNK)
        def _(i):
            @pl.loop(0, dh, step=16)
            def _(c):
                rows[pl.ds(i, 1), pl.ds(c, 16)] = jnp.zeros((1, 16),
                                                            jnp.float32)

        @pl.loop(0, nrt, step=_CHUNK)
        def _(r):
            pltpu.sync_copy(rows.at[pl.ds(0, _CHUNK)],
                            z_sh.at[pl.ds(sid * nrt + r, _CHUNK)])
            pltpu.sync_copy(y_hbm.at[cid].at[pl.ds(sid * nrt + r, _CHUNK)],
                            rows.at[pl.ds(_CHUNK, _CHUNK)])
            pltpu.sync_copy(rows.at[pl.ds(_CHUNK, _CHUNK)],
                            y_sp.at[pl.ds(sid * nrt + r, _CHUNK)])

        plsc.subcore_barrier()

        for j in range(4):
            idx_cp(j, j).start()
        idx_cp(0, 0).wait()
        g_cp(0, 0).start()

        @pl.loop(0, nch, step=4)
        def _(c0):
            for j in range(4):
                c = c0 + j
                jn = (j + 1) % 4
                g_cp(c, j).wait()
                cn = c + 1

                @pl.when(cn < nch)
                def _():
                    idx_cp(cn, jn).wait()
                    g_cp(cn, jn).start()

                pltpu.sync_copy(rows.at[pl.ds(j * _CHUNK, _CHUNK)],
                                z_sh.at[idxr.at[j, 1]], add=True)

                @pl.when(c + 4 < nch)
                def _():
                    idx_cp(c + 4, j).start()

        plsc.subcore_barrier()
        for kk in range(nrt // _CHUNK):
            off = sid * nrt + kk * _CHUNK
            pltpu.sync_copy(z_sh.at[pl.ds(off, _CHUNK)], rows.at[pl.ds(0, _CHUNK)])
            pltpu.sync_copy(rows.at[pl.ds(0, _CHUNK)],
                            out_hbm.at[cid].at[pl.ds(off, _CHUNK)])

    return k(y2, sd4)


def _dense_in_tc(x, W_in, b_in, ln1_w, ln1_b, conv_W, n, n_pad):
    d = x.shape[1]
    dh = W_in.shape[1]
    blk = 1024
    grid = n_pad // blk

    def body(x_ref, w1_ref, b1_ref, lw_ref, lb_ref, w2_ref, h_ref, xw_ref):
        i = pl.program_id(0)
        t = jnp.dot(x_ref[...], w1_ref[...], preferred_element_type=jnp.float32)
        t = t + b1_ref[...]
        mu = jnp.mean(t, axis=1, keepdims=True)
        var = jnp.mean((t - mu) ** 2, axis=1, keepdims=True)
        t = (t - mu) * lax.rsqrt(var + 1e-5) * lw_ref[...] + lb_ref[...]
        h = jnp.maximum(t, 0.0)
        rowid = i * blk + lax.broadcasted_iota(jnp.int32, (blk, 1), 0)
        h = jnp.where(rowid < n, h, 0.0)
        h_ref[...] = h
        xw_ref[...] = jnp.dot(h, w2_ref[...], preferred_element_type=jnp.float32)

    return pl.pallas_call(
        body,
        grid=(grid,),
        in_specs=[
            pl.BlockSpec((blk, d), lambda i: (i, 0)),
            pl.BlockSpec((d, dh), lambda i: (0, 0)),
            pl.BlockSpec((1, dh), lambda i: (0, 0)),
            pl.BlockSpec((1, dh), lambda i: (0, 0)),
            pl.BlockSpec((1, dh), lambda i: (0, 0)),
            pl.BlockSpec((dh, dh), lambda i: (0, 0)),
        ],
        out_specs=[pl.BlockSpec((blk, dh), lambda i: (i, 0))] * 2,
        out_shape=[jax.ShapeDtypeStruct((n_pad, dh), jnp.float32)] * 2,
    )(x, W_in.astype(jnp.float32), b_in.reshape(1, dh), ln1_w.reshape(1, dh),
      ln1_b.reshape(1, dh), conv_W.astype(jnp.float32))


def _y_tc(cnt, xw_pad, n_pad):
    d = xw_pad.shape[1]
    dh = d // 2
    blk = 1024
    grid = n_pad // blk

    def body(cnt_ref, xw_ref, y_ref, dv_ref):
        c = cnt_ref[...]
        deg = c[0] + c[1] + 1.0
        dinv = lax.rsqrt(deg)
        dv_ref[...] = dinv
        y = xw_ref[...] * dinv
        y_ref[...] = jnp.stack([y[:, :dh], y[:, dh:]], axis=0)

    return pl.pallas_call(
        body,
        grid=(grid,),
        in_specs=[
            pl.BlockSpec((2, blk, 1), lambda i: (0, i, 0)),
            pl.BlockSpec((blk, d), lambda i: (i, 0)),
        ],
        out_specs=[
            pl.BlockSpec((2, blk, dh), lambda i: (0, i, 0)),
            pl.BlockSpec((blk, 1), lambda i: (i, 0)),
        ],
        out_shape=[
            jax.ShapeDtypeStruct((2, n_pad, dh), jnp.float32),
            jax.ShapeDtypeStruct((n_pad, 1), jnp.float32),
        ],
    )(cnt, xw_pad)


def _b1_tc(zp, xw_pad, dinv, conv_b, n, n_pad):
    d = xw_pad.shape[1]
    dh = d // 2
    blk = 1000
    grid = n // blk

    def body(z_ref, xw_ref, dv_ref, cb_ref, agg_ref, s_ref):
        i = pl.program_id(0)
        zs = z_ref[...]
        z = jnp.concatenate([zs[0], zs[1]], axis=1)
        dv = dv_ref[...]
        agg = dv * z + dv * dv * xw_ref[...] + cb_ref[...]
        agg_ref[...] = agg
        s1 = jnp.sum(agg, axis=0, keepdims=True)
        s2 = jnp.sum(agg * agg, axis=0, keepdims=True)
        s = jnp.concatenate([s1, s2], axis=0)

        @pl.when(i == 0)
        def _():
            s_ref[...] = s

        @pl.when(i > 0)
        def _():
            s_ref[...] += s

    return pl.pallas_call(
        body,
        grid=(grid,),
        in_specs=[
            pl.BlockSpec((2, blk, dh), lambda i: (0, i, 0)),
            pl.BlockSpec((blk, d), lambda i: (i, 0)),
            pl.BlockSpec((blk, 1), lambda i: (i, 0)),
            pl.BlockSpec((1, d), lambda i: (0, 0)),
        ],
        out_specs=[
            pl.BlockSpec((blk, d), lambda i: (i, 0)),
            pl.BlockSpec((2, d), lambda i: (0, 0)),
        ],
        out_shape=[
            jax.ShapeDtypeStruct((n, d), jnp.float32),
            jax.ShapeDtypeStruct((2, d), jnp.float32),
        ],
    )(zp, xw_pad, dinv, conv_b.reshape(1, d))


def _b2_tc(agg, s, h_pad, gn_w, gn_b, gn_ms, proj_W, proj_b, ln2_w, ln2_b, n):
    d = agg.shape[1]
    do = proj_W.shape[1]
    blk = 1000
    grid = n // blk

    def body(agg_ref, s_ref, h_ref, gw_ref, gb_ref, gms_ref, pw_ref, pb_ref,
             l2w_ref, l2b_ref, out_ref):
        s_val = s_ref[...]
        mean = s_val[0:1] / n
        m = gms_ref[...] * mean
        var = s_val[1:2] / n - 2.0 * m * mean + m * m
        c = (agg_ref[...] - m) * lax.rsqrt(var + 1e-5) * gw_ref[...] + gb_ref[...]
        c = jnp.maximum(c, 0.0)
        hr = c + h_ref[...]
        p = jnp.dot(hr, pw_ref[...], preferred_element_type=jnp.float32)
        p = p + pb_ref[...]
        mu = jnp.mean(p, axis=1, keepdims=True)
        v2 = jnp.mean((p - mu) ** 2, axis=1, keepdims=True)
        p = (p - mu) * lax.rsqrt(v2 + 1e-5) * l2w_ref[...] + l2b_ref[...]
        nrm = jnp.sqrt(jnp.sum(p * p, axis=1, keepdims=True))
        out_ref[...] = p / jnp.maximum(nrm, 1e-12)

    return pl.pallas_call(
        body,
        grid=(grid,),
        in_specs=[
            pl.BlockSpec((blk, d), lambda i: (i, 0)),
            pl.BlockSpec((2, d), lambda i: (0, 0)),
            pl.BlockSpec((blk, d), lambda i: (i, 0)),
            pl.BlockSpec((1, d), lambda i: (0, 0)),
            pl.BlockSpec((1, d), lambda i: (0, 0)),
            pl.BlockSpec((1, d), lambda i: (0, 0)),
            pl.BlockSpec((d, do), lambda i: (0, 0)),
            pl.BlockSpec((1, do), lambda i: (0, 0)),
            pl.BlockSpec((1, do), lambda i: (0, 0)),
            pl.BlockSpec((1, do), lambda i: (0, 0)),
        ],
        out_specs=pl.BlockSpec((blk, do), lambda i: (i, 0)),
        out_shape=jax.ShapeDtypeStruct((n, do), jnp.float32),
    )(agg, s, h_pad[:n], gn_w.reshape(1, d), gn_b.reshape(1, d),
      gn_ms.reshape(1, d), proj_W, proj_b.reshape(1, do),
      ln2_w.reshape(1, do), ln2_b.reshape(1, do))


def kernel(x, edge_index, W_in, b_in, ln1_w, ln1_b, conv_W, conv_b,
           gn_w, gn_b, gn_ms, proj_W, proj_b, ln2_w, ln2_b):
    n = x.shape[0]
    e = edge_index.shape[1]
    n_pad = ((n + 1023) // 1024) * 1024
    ept = ((e + _NS * _GCH - 1) // (_NS * _GCH)) * _GCH
    e_pad = ept * _NS
    nch = ept // _CHUNK

    s = edge_index[0]
    dst = edge_index[1]
    pad = e_pad - e
    s_pad = jnp.concatenate([s, jnp.full((pad,), n, jnp.int32)])
    d_pad = jnp.concatenate([dst, jnp.full((pad,), n, jnp.int32)])
    sd4 = jnp.stack([s_pad.reshape(_NS, nch, _CHUNK),
                     d_pad.reshape(_NS, nch, _CHUNK)], axis=2)

    epth = e_pad // (_NC * _NS)
    d3h = d_pad.reshape(_NC * _NS, epth // _CHUNK, _CHUNK)

    cnt_w = _hist_sc(d3h, n_pad, epth)
    cnt = cnt_w[:, :, :1]

    h_pad, xw_pad = _dense_in_tc(x, W_in, b_in, ln1_w, ln1_b, conv_W, n, n_pad)
    y2, dinv = _y_tc(cnt, xw_pad, n_pad)
    zp = _scatter_sc(y2, sd4, n_pad, ept)
    agg, s_mom = _b1_tc(zp, xw_pad, dinv, conv_b, n, n_pad)
    return _b2_tc(agg, s_mom, h_pad, gn_w, gn_b, gn_ms, proj_W, proj_b,
                  ln2_w, ln2_b, n)
